# Initial kernel scaffold; baseline (speedup 1.0000x reference)
import jax
import jax.numpy as jnp
from jax.experimental import pallas as pl


def kernel(registration_pred, registration_gt, coords, wandb):
    raise NotImplementedError("write your pallas kernel here")


# speedup vs baseline: 12.4673x; 12.4673x over previous
import functools

import jax
import jax.numpy as jnp
from jax import lax
from jax.experimental import pallas as pl
from jax.experimental.pallas import tpu as pltpu
from jax.experimental.pallas import tpu_sc as plsc

_N = 131072
_D = _H = _W = 128
_NC = 2
_NS = 16
_CHUNK = 2048
_PER_TILE = _N // _NS
_SUBCHUNKS = _PER_TILE // _CHUNK
_HALF_WORDS = (_D // _NC) * _H * _W
_TILE_WORDS = _HALF_WORDS // _NS
_GROUPS = _CHUNK // 16
_ZCHUNK = 16384


def _floor_i32(f):
    t = f.astype(jnp.int32)
    tf = t.astype(jnp.float32)
    return jnp.where(tf > f, t - 1, t)


def _axis_terms(coord, lim):
    f = ((coord + 1.0) * 128.0 - 1.0) * 0.5
    c0 = _floor_i32(f)
    t = f - c0.astype(jnp.float32)
    v0 = (c0 >= 0) & (c0 <= lim - 1)
    v1 = (c0 >= -1) & (c0 <= lim - 2)
    a0 = jnp.where(v0, 1.0 - t, 0.0)
    a1 = jnp.where(v1, t, 0.0)
    i0 = jnp.clip(c0, 0, lim - 1)
    i1 = jnp.clip(c0 + 1, 0, lim - 1)
    return a0, a1, i0, i1


def _sc_body(px_hbm, py_hbm, pz_hbm, gx_hbm, gy_hbm, gz_hbm,
             cx_hbm, cy_hbm, cz_hbm, out_hbm,
             pxb, pyb, pzb, cxb, cyb, czb, idx_st, val_st, zb, accv, grid,
             in_sem, scat_sem):
    pb = (pxb, pyb, pzb)
    cb = (cxb, cyb, czb)
    pred_hbm = (px_hbm, py_hbm, pz_hbm)
    gt_hbm = (gx_hbm, gy_hbm, gz_hbm)
    coords_hbm = (cx_hbm, cy_hbm, cz_hbm)
    ci = lax.axis_index("c")
    si = lax.axis_index("s")
    zoff = ci * (_D // _NC)
    tile_base = si * _TILE_WORDS

    zeros16 = jnp.zeros((16,), jnp.float32)

    def _zfill(t, carry):
        zb[pl.ds(t * 16, 16)] = zeros16
        return carry
    lax.fori_loop(0, _ZCHUNK // 16, _zfill, 0)
    for k in range(_TILE_WORDS // _ZCHUNK):
        pltpu.sync_copy(zb, grid.at[pl.ds(tile_base + k * _ZCHUNK, _ZCHUNK)])
    plsc.subcore_barrier()

    for cloud, (src_hbm, sign) in enumerate(((pred_hbm, 1.0), (gt_hbm, -1.0))):
        def _chunk_body(sub, carry, src_hbm=src_hbm, sign=sign):
            base = si * _PER_TILE + sub * _CHUNK
            for d in range(3):
                pltpu.async_copy(src_hbm[d].at[pl.ds(base, _CHUNK)], pb[d],
                                 in_sem)
                pltpu.async_copy(coords_hbm[d].at[pl.ds(base, _CHUNK)], cb[d],
                                 in_sem)
            pltpu.make_async_copy(px_hbm.at[pl.ds(0, 6 * _CHUNK)],
                                  zb.at[pl.ds(0, 6 * _CHUNK)], in_sem).wait()

            def _group(i, c2):
                s16 = pl.ds(i * 16, 16)
                x = pb[0][s16] + cb[0][s16]
                y = pb[1][s16] + cb[1][s16]
                z = pb[2][s16] + cb[2][s16]
                ax0, ax1, xi0, xi1 = _axis_terms(x, _W)
                ay0, ay1, yi0, yi1 = _axis_terms(y, _H)
                fz = ((z + 1.0) * 128.0 - 1.0) * 0.5
                z0 = _floor_i32(fz)
                tz = fz - z0.astype(jnp.float32)
                z0l = z0 - zoff
                vz0 = (z0l >= 0) & (z0l <= 63)
                vz1 = (z0l >= -1) & (z0l <= 62)
                az0 = jnp.where(vz0, (1.0 - tz) * sign, 0.0)
                az1 = jnp.where(vz1, tz * sign, 0.0)
                zb0 = jnp.clip(z0l, 0, 63) * (_H * _W)
                zb1 = jnp.clip(z0l + 1, 0, 63) * (_H * _W)

                yb0 = yi0 * _W
                yb1 = yi1 * _W
                b00 = zb0 + yb0
                b01 = zb0 + yb1
                b10 = zb1 + yb0
                b11 = zb1 + yb1
                a00 = az0 * ay0
                a01 = az0 * ay1
                a10 = az1 * ay0
                a11 = az1 * ay1
                corners = (
                    (b00, a00), (b01, a01), (b10, a10), (b11, a11))
                for k2, (bzy, azy) in enumerate(corners):
                    idx_st[i, pl.ds((2 * k2) * 16, 16)] = bzy + xi0
                    val_st[i, pl.ds((2 * k2) * 16, 16)] = azy * ax0
                    idx_st[i, pl.ds((2 * k2 + 1) * 16, 16)] = bzy + xi1
                    val_st[i, pl.ds((2 * k2 + 1) * 16, 16)] = azy * ax1
                return c2
            lax.fori_loop(0, _GROUPS, _group, 0)

            def _scat(j, c3):
                pltpu.async_copy(val_st.at[j], grid.at[idx_st.at[j]],
                                 scat_sem, add=True)
                return c3
            lax.fori_loop(0, _GROUPS, _scat, 0)
            pltpu.make_async_copy(px_hbm.at[pl.ds(0, _ZCHUNK)], zb,
                                  scat_sem).wait()
            return carry
        lax.fori_loop(0, _SUBCHUNKS, _chunk_body, 0)

    plsc.subcore_barrier()

    def _hchunk(k, acc):
        pltpu.sync_copy(grid.at[pl.ds(tile_base + k * _ZCHUNK, _ZCHUNK)], zb)

        def _hstep(t, a):
            dv = zb[pl.ds(t * 16, 16)]
            ad = jnp.abs(dv)
            return a + jnp.where(ad < 1.0, 0.5 * dv * dv, ad - 0.5)
        return lax.fori_loop(0, _ZCHUNK // 16, _hstep, acc)
    acc = lax.fori_loop(0, _TILE_WORDS // _ZCHUNK, _hchunk, zeros16)
    accv[...] = acc
    pltpu.sync_copy(accv, out_hbm.at[ci, si])


@jax.jit
def _divroc_sc(px, py, pz, gx, gy, gz, cx, cy, cz):
    mesh = plsc.VectorSubcoreMesh(
        core_axis_name="c", subcore_axis_name="s",
        num_cores=_NC, num_subcores=_NS)
    fn = pl.kernel(
        _sc_body,
        out_type=jax.ShapeDtypeStruct((_NC, _NS, 16), jnp.float32),
        mesh=mesh,
        scratch_types=[
            pltpu.VMEM((_CHUNK,), jnp.float32),
            pltpu.VMEM((_CHUNK,), jnp.float32),
            pltpu.VMEM((_CHUNK,), jnp.float32),
            pltpu.VMEM((_CHUNK,), jnp.float32),
            pltpu.VMEM((_CHUNK,), jnp.float32),
            pltpu.VMEM((_CHUNK,), jnp.float32),
            pltpu.VMEM((_GROUPS, 128), jnp.int32),
            pltpu.VMEM((_GROUPS, 128), jnp.float32),
            pltpu.VMEM((_ZCHUNK,), jnp.float32),
            pltpu.VMEM((16,), jnp.float32),
            pltpu.VMEM_SHARED((_HALF_WORDS,), jnp.float32),
            pltpu.SemaphoreType.DMA,
            pltpu.SemaphoreType.DMA,
        ],
    )
    return fn(px, py, pz, gx, gy, gz, cx, cy, cz)


def kernel(registration_pred, registration_gt, coords, wandb):
    n = registration_pred.shape[1]
    p = registration_pred.reshape(n, 3)
    g = registration_gt.reshape(n, 3)
    c = coords.reshape(n, 3)
    parts = _divroc_sc(p[:, 0], p[:, 1], p[:, 2],
                       g[:, 0], g[:, 1], g[:, 2],
                       c[:, 0], c[:, 1], c[:, 2])
    return jnp.sum(parts)
